# Initial kernel scaffold; baseline (speedup 1.0000x reference)
import functools

import jax
import jax.numpy as jnp
from jax import lax
from jax.experimental import pallas as pl
from jax.experimental.pallas import tpu as pltpu
from jax.experimental.pallas import tpu_sc as plsc

B = 1024
NH = 50
NN = 5
D = 128
AD = 64
CAT = 18
CATP = 19
TL = 20
V = 30000
WD = 64
NODES = NH + CAT
SCALE = 8.0


NC = 2
NS = 16
NW = NC * NS
NPAIR = B * NH
PAIRS_PER_W = NPAIR // NW
CP = 32
NCHUNK = PAIRS_PER_W // CP
IDX_CHUNK = CP * TL
NGATHER = IDX_CHUNK // 128

_sc_mesh = plsc.VectorSubcoreMesh(core_axis_name="c", subcore_axis_name="s")


@functools.partial(
    pl.kernel,
    mesh=_sc_mesh,
    out_type=jax.ShapeDtypeStruct((NPAIR, WD), jnp.float32),
    scratch_types=[
        pltpu.VMEM((NGATHER, 128), jnp.int32),
        pltpu.VMEM((NGATHER, 128), jnp.float32),
        pltpu.VMEM((IDX_CHUNK, WD), jnp.float32),
        pltpu.VMEM((CP, WD), jnp.float32),
        pltpu.SemaphoreType.DMA,
    ],
)
def _pool_sc(idx_hbm, mask_hbm, table_hbm, out_hbm, idx_v, mask_v, rows_v,
             acc_v, sem):
    wid = lax.axis_index("s") * NC + lax.axis_index("c")
    pair_base = wid * PAIRS_PER_W

    def chunk_body(ci, carry):
        pbase = pair_base + ci * CP
        rbase = pbase * TL // 128
        pltpu.sync_copy(idx_hbm.at[pl.ds(rbase, NGATHER)], idx_v)
        pltpu.sync_copy(mask_hbm.at[pl.ds(rbase, NGATHER)], mask_v)
        for j in range(NGATHER):
            for i in range(8):
                m = mask_v[j, pl.ds(i * 16, 16)]
                iv = idx_v[j, pl.ds(i * 16, 16)]
                idx_v[j, pl.ds(i * 16, 16)] = jnp.where(m > 0.0, iv, V)
        copies = [
            pltpu.async_copy(table_hbm.at[idx_v.at[j]],
                             rows_v.at[pl.ds(j * 128, 128)], sem)
            for j in range(NGATHER)
        ]
        for c in copies:
            c.wait()

        def pair_body(p, c2):
            rb = p * TL
            for k in range(WD // 16):
                s = rows_v[rb, pl.ds(k * 16, 16)]
                for j in range(1, TL):
                    s = s + rows_v[rb + j, pl.ds(k * 16, 16)]
                acc_v[p, pl.ds(k * 16, 16)] = s
            return c2

        lax.fori_loop(0, CP, pair_body, 0)
        pltpu.sync_copy(acc_v, out_hbm.at[pl.ds(pbase, CP)])
        return carry

    lax.fori_loop(0, NCHUNK, chunk_body, 0)


BB = 8


def _dense_body(sums_ref, tmask_ref, gidx_ref, cmask_ref, graph_ref, cand_ref,
                Wn_ref, bn_ref, proxy_ref, W0_ref, b0_ref, W1_ref, b1_ref,
                Kw_ref, Qw_ref, Qb_ref, aW_ref, ab_ref, iKw_ref, iQw_ref,
                iQb_ref, out_ref):
    f32 = jnp.float32
    cnt = jnp.sum(tmask_ref[...], axis=2)
    pooled = sums_ref[...] / jnp.maximum(cnt, 1e-6)[..., None]
    hist = (pooled.reshape(BB * NH, WD) @ Wn_ref[...]
            + bn_ref[...])
    proxy = proxy_ref[...]
    W0 = W0_ref[...]
    W1 = W1_ref[...]
    b0 = b0_ref[...]
    b1 = b1_ref[...]
    Kw = Kw_ref[...]
    Qw = Qw_ref[...]
    Qb = Qb_ref[...]
    aW = aW_ref[...]
    ab = ab_ref[...]
    iKw = iKw_ref[...]
    iQw = iQw_ref[...]
    iQb = iQb_ref[...]
    cat_iota = lax.broadcasted_iota(jnp.int32, (NH, CATP), 1)

    for s in range(BB):
        h0 = jnp.concatenate([hist[s * NH:(s + 1) * NH], proxy], axis=0)
        A = graph_ref[s]
        t0 = A @ h0
        h1 = jax.nn.relu(t0 @ W0 + b0) + h0
        t1 = A @ h1
        h2 = t1 @ W1 + b1 + h1
        g = (h2 + h0)[:NH]

        K = g @ Kw
        cand = cand_ref[s]
        Q = cand @ Qw + Qb
        a = lax.dot_general(Q, K, (((1,), (1,)), ((), ()))) / SCALE

        idx = gidx_ref[s]
        oh = (idx[:, None] == cat_iota).astype(f32)
        seg_max = jnp.max(
            jnp.where(oh[None, :, :] > 0, a[:, :, None], -1e9), axis=1)
        max_g = lax.dot_general(seg_max, oh, (((1,), (1,)), ((), ())))
        exp_a = jnp.exp(a - max_g)
        seg_sum = exp_a @ oh
        denom = lax.dot_general(seg_sum, oh, (((1,), (1,)), ((), ())))
        alpha = exp_a / denom

        M = (alpha[:, None, :] * oh.T[None, :, :]).reshape(NN * CATP, NH)
        intra = M @ g
        intra = jax.nn.relu(intra @ aW + ab) + intra

        Kf = (intra @ iKw).reshape(NN, CATP, AD)
        Qf = cand @ iQw + iQb
        satt = jnp.sum(Kf * Qf[:, None, :], axis=2) / SCALE
        cm = cmask_ref[s]
        cm = jnp.where(lax.iota(jnp.int32, CATP) == CATP - 1, 1.0, cm)
        satt = jnp.where(cm[None, :] == 0, -1e9, satt)
        satt = satt - jnp.max(satt, axis=1, keepdims=True)
        e = jnp.exp(satt)
        al = e / jnp.sum(e, axis=1, keepdims=True)
        out_ref[s] = jnp.sum(
            al[:, :, None] * intra.reshape(NN, CATP, D), axis=1)


def _full(shape):
    return pl.BlockSpec(shape, lambda i: (0,) * len(shape))


_dense = pl.pallas_call(
    _dense_body,
    grid=(B // BB,),
    in_specs=[
        pl.BlockSpec((BB, NH, WD), lambda i: (i, 0, 0)),
        pl.BlockSpec((BB, NH, TL), lambda i: (i, 0, 0)),
        pl.BlockSpec((BB, NH), lambda i: (i, 0)),
        pl.BlockSpec((BB, CATP), lambda i: (i, 0)),
        pl.BlockSpec((BB, NODES, NODES), lambda i: (i, 0, 0)),
        pl.BlockSpec((BB, NN, D), lambda i: (i, 0, 0)),
        _full((WD, D)),
        _full((1, D)),
        _full((CAT, D)),
        _full((D, D)),
        _full((1, D)),
        _full((D, D)),
        _full((1, D)),
        _full((D, AD)),
        _full((D, AD)),
        _full((1, AD)),
        _full((D, D)),
        _full((1, D)),
        _full((D, AD)),
        _full((D, AD)),
        _full((1, AD)),
    ],
    out_specs=pl.BlockSpec((BB, NN, D), lambda i: (i, 0, 0)),
    out_shape=jax.ShapeDtypeStruct((B, NN, D), jnp.float32),
    compiler_params=pltpu.CompilerParams(
        dimension_semantics=("arbitrary",)),
)


def kernel(user_title_text, user_title_mask, user_title_entity,
           user_content_text, user_content_mask, user_content_entity,
           user_category, user_subCategory, user_history_mask,
           user_history_graph, user_history_category_mask,
           user_history_category_indices, user_embedding,
           candidate_news_representation, word_emb, W_news, b_news, proxy_emb,
           gcn_W0, gcn_b0, gcn_W1, gcn_b1, Kw, Qw, Qb, aff_W, aff_b, inter_Kw,
           inter_Qw, inter_Qb):
    idx2d = user_title_text.astype(jnp.int32).reshape(-1, 128)
    mask2d = user_title_mask.reshape(-1, 128)
    table_z = jnp.concatenate(
        [word_emb, jnp.zeros((1, WD), jnp.float32)], axis=0)
    sums = _pool_sc(idx2d, mask2d, table_z)
    out = _dense(
        sums.reshape(B, NH, WD),
        user_title_mask,
        user_history_category_indices.astype(jnp.int32),
        user_history_category_mask,
        user_history_graph,
        candidate_news_representation,
        W_news,
        b_news.reshape(1, D),
        proxy_emb,
        gcn_W0,
        gcn_b0.reshape(1, D),
        gcn_W1,
        gcn_b1.reshape(1, D),
        Kw,
        Qw,
        Qb.reshape(1, AD),
        aff_W,
        aff_b.reshape(1, D),
        inter_Kw,
        inter_Qw,
        inter_Qb.reshape(1, AD),
    )
    return out


# speedup vs baseline: 1.5188x; 1.5188x over previous
import functools

import jax
import jax.numpy as jnp
from jax import lax
from jax.experimental import pallas as pl
from jax.experimental.pallas import tpu as pltpu
from jax.experimental.pallas import tpu_sc as plsc

B = 1024
NH = 50
NN = 5
D = 128
AD = 64
CAT = 18
CATP = 19
TL = 20
V = 30000
WD = 64
NODES = NH + CAT
SCALE = 8.0


NC = 2
NS = 16
NW = NC * NS
NPAIR = B * NH
PAIRS_PER_W = NPAIR // NW
CP = 32
NCHUNK = PAIRS_PER_W // CP
IDX_CHUNK = CP * TL
NGATHER = IDX_CHUNK // 128

def _pool_sc_body(idx_hbm, mask_hbm, table_hbm, out_hbm, idx_v, mask_v, rows_v,
                  acc_v, sem):
    wid = lax.axis_index("s") * NC + lax.axis_index("c")
    pair_base = wid * PAIRS_PER_W

    def chunk_body(ci, carry):
        pbase = pair_base + ci * CP
        ibase = pbase * TL
        for j in range(NGATHER):
            pltpu.sync_copy(idx_hbm.at[pl.ds(ibase + j * 128, 128)],
                            idx_v.at[j])
            pltpu.sync_copy(mask_hbm.at[pl.ds(ibase + j * 128, 128)],
                            mask_v.at[j])
        for j in range(NGATHER):
            for i in range(8):
                m = mask_v[j, pl.ds(i * 16, 16)]
                iv = idx_v[j, pl.ds(i * 16, 16)]
                idx_v[j, pl.ds(i * 16, 16)] = jnp.where(m > 0.0, iv, V)
        copies = [
            pltpu.async_copy(table_hbm.at[idx_v.at[j]],
                             rows_v.at[pl.ds(j * 128, 128)], sem)
            for j in range(NGATHER)
        ]
        for c in copies:
            c.wait()

        def pair_body(p, c2):
            rb = p * TL
            for k in range(WD // 16):
                s = rows_v[rb, pl.ds(k * 16, 16)]
                for j in range(1, TL):
                    s = s + rows_v[rb + j, pl.ds(k * 16, 16)]
                acc_v[p, pl.ds(k * 16, 16)] = s
            return c2

        lax.fori_loop(0, CP, pair_body, 0)
        pltpu.sync_copy(acc_v, out_hbm.at[pl.ds(pbase, CP)])
        return carry

    lax.fori_loop(0, NCHUNK, chunk_body, 0)


@functools.cache
def _make_pool_sc():
    mesh = plsc.VectorSubcoreMesh(core_axis_name="c", subcore_axis_name="s")
    return pl.kernel(
        _pool_sc_body,
        mesh=mesh,
        out_type=jax.ShapeDtypeStruct((NPAIR, WD), jnp.float32),
        scratch_types=[
            pltpu.VMEM((NGATHER, 128), jnp.int32),
            pltpu.VMEM((NGATHER, 128), jnp.float32),
            pltpu.VMEM((IDX_CHUNK, WD), jnp.float32),
            pltpu.VMEM((CP, WD), jnp.float32),
            pltpu.SemaphoreType.DMA,
        ],
        compiler_params=pltpu.CompilerParams(use_tc_tiling_on_sc=False),
    )


BB = 8


def _dense_body(sums_ref, tmask_ref, gidx_ref, cmask_ref, graph_ref, cand_ref,
                Wn_ref, bn_ref, proxy_ref, W0_ref, b0_ref, W1_ref, b1_ref,
                Kw_ref, Qw_ref, Qb_ref, aW_ref, ab_ref, iKw_ref, iQw_ref,
                iQb_ref, out_ref):
    f32 = jnp.float32
    cnt = jnp.sum(tmask_ref[...], axis=2)
    pooled = sums_ref[...] / jnp.maximum(cnt, 1e-6)[..., None]
    hist = (pooled.reshape(BB * NH, WD) @ Wn_ref[...]
            + bn_ref[...])
    proxy = proxy_ref[...]
    W0 = W0_ref[...]
    W1 = W1_ref[...]
    b0 = b0_ref[...]
    b1 = b1_ref[...]
    Kw = Kw_ref[...]
    Qw = Qw_ref[...]
    Qb = Qb_ref[...]
    aW = aW_ref[...]
    ab = ab_ref[...]
    iKw = iKw_ref[...]
    iQw = iQw_ref[...]
    iQb = iQb_ref[...]
    cat_iota = lax.broadcasted_iota(jnp.int32, (NH, CATP), 1)

    for s in range(BB):
        h0 = jnp.concatenate([hist[s * NH:(s + 1) * NH], proxy], axis=0)
        A = graph_ref[s]
        t0 = A @ h0
        h1 = jax.nn.relu(t0 @ W0 + b0) + h0
        t1 = A @ h1
        h2 = t1 @ W1 + b1 + h1
        g = (h2 + h0)[:NH]

        K = g @ Kw
        cand = cand_ref[s]
        Q = cand @ Qw + Qb
        a = lax.dot_general(Q, K, (((1,), (1,)), ((), ()))) / SCALE

        idx = gidx_ref[s]
        oh = (idx[:, None] == cat_iota).astype(f32)
        seg_max = jnp.max(
            jnp.where(oh[None, :, :] > 0, a[:, :, None], -1e9), axis=1)
        max_g = lax.dot_general(seg_max, oh, (((1,), (1,)), ((), ())))
        exp_a = jnp.exp(a - max_g)
        seg_sum = exp_a @ oh
        denom = lax.dot_general(seg_sum, oh, (((1,), (1,)), ((), ())))
        alpha = exp_a / denom

        M = (alpha[:, None, :] * oh.T[None, :, :]).reshape(NN * CATP, NH)
        intra = M @ g
        intra = jax.nn.relu(intra @ aW + ab) + intra

        Kf = (intra @ iKw).reshape(NN, CATP, AD)
        Qf = cand @ iQw + iQb
        satt = jnp.sum(Kf * Qf[:, None, :], axis=2) / SCALE
        cm = cmask_ref[s]
        cm = jnp.where(lax.iota(jnp.int32, CATP) == CATP - 1, 1.0, cm)
        satt = jnp.where(cm[None, :] == 0, -1e9, satt)
        satt = satt - jnp.max(satt, axis=1, keepdims=True)
        e = jnp.exp(satt)
        al = e / jnp.sum(e, axis=1, keepdims=True)
        out_ref[s] = jnp.sum(
            al[:, :, None] * intra.reshape(NN, CATP, D), axis=1)


def _full(shape):
    return pl.BlockSpec(shape, lambda i: (0,) * len(shape))


_dense = pl.pallas_call(
    _dense_body,
    grid=(B // BB,),
    in_specs=[
        pl.BlockSpec((BB, NH, WD), lambda i: (i, 0, 0)),
        pl.BlockSpec((BB, NH, TL), lambda i: (i, 0, 0)),
        pl.BlockSpec((BB, NH), lambda i: (i, 0)),
        pl.BlockSpec((BB, CATP), lambda i: (i, 0)),
        pl.BlockSpec((BB, NODES, NODES), lambda i: (i, 0, 0)),
        pl.BlockSpec((BB, NN, D), lambda i: (i, 0, 0)),
        _full((WD, D)),
        _full((1, D)),
        _full((CAT, D)),
        _full((D, D)),
        _full((1, D)),
        _full((D, D)),
        _full((1, D)),
        _full((D, AD)),
        _full((D, AD)),
        _full((1, AD)),
        _full((D, D)),
        _full((1, D)),
        _full((D, AD)),
        _full((D, AD)),
        _full((1, AD)),
    ],
    out_specs=pl.BlockSpec((BB, NN, D), lambda i: (i, 0, 0)),
    out_shape=jax.ShapeDtypeStruct((B, NN, D), jnp.float32),
    compiler_params=pltpu.CompilerParams(
        dimension_semantics=("arbitrary",)),
)


def kernel(user_title_text, user_title_mask, user_title_entity,
           user_content_text, user_content_mask, user_content_entity,
           user_category, user_subCategory, user_history_mask,
           user_history_graph, user_history_category_mask,
           user_history_category_indices, user_embedding,
           candidate_news_representation, word_emb, W_news, b_news, proxy_emb,
           gcn_W0, gcn_b0, gcn_W1, gcn_b1, Kw, Qw, Qb, aff_W, aff_b, inter_Kw,
           inter_Qw, inter_Qb):
    idx2d = user_title_text.astype(jnp.int32).reshape(-1)
    mask2d = user_title_mask.reshape(-1)
    table_z = jnp.concatenate(
        [word_emb, jnp.zeros((1, WD), jnp.float32)], axis=0)
    sums = _make_pool_sc()(idx2d, mask2d, table_z)
    out = _dense(
        sums.reshape(B, NH, WD),
        user_title_mask,
        user_history_category_indices.astype(jnp.int32),
        user_history_category_mask,
        user_history_graph,
        candidate_news_representation,
        W_news,
        b_news.reshape(1, D),
        proxy_emb,
        gcn_W0,
        gcn_b0.reshape(1, D),
        gcn_W1,
        gcn_b1.reshape(1, D),
        Kw,
        Qw,
        Qb.reshape(1, AD),
        aff_W,
        aff_b.reshape(1, D),
        inter_Kw,
        inter_Qw,
        inter_Qb.reshape(1, AD),
    )
    return out


# speedup vs baseline: 1.5197x; 1.0006x over previous
import functools

import jax
import jax.numpy as jnp
from jax import lax
from jax.experimental import pallas as pl
from jax.experimental.pallas import tpu as pltpu
from jax.experimental.pallas import tpu_sc as plsc

B = 1024
NH = 50
NN = 5
D = 128
AD = 64
CAT = 18
CATP = 19
TL = 20
V = 30000
WD = 64
NODES = NH + CAT
SCALE = 8.0


NC = 2
NS = 16
NW = NC * NS
NPAIR = B * NH
PAIRS_PER_W = NPAIR // NW
CP = 32
NCHUNK = PAIRS_PER_W // CP
IDX_CHUNK = CP * TL
NGATHER = IDX_CHUNK // 128

IDX_ROWS = PAIRS_PER_W * TL // 128


def _pool_sc_body(idx_hbm, mask_hbm, table_hbm, out_hbm, idx_v, mask_v, rows_v,
                  acc_v, sem):
    wid = lax.axis_index("s") * NC + lax.axis_index("c")
    pair_base = wid * PAIRS_PER_W

    pltpu.sync_copy(idx_hbm.at[wid], idx_v)
    pltpu.sync_copy(mask_hbm.at[wid], mask_v)

    def sel_body(r, carry):
        for i in range(8):
            m = mask_v[r, pl.ds(i * 16, 16)]
            iv = idx_v[r, pl.ds(i * 16, 16)]
            idx_v[r, pl.ds(i * 16, 16)] = jnp.where(m > 0.0, iv, V)
        return carry

    lax.fori_loop(0, IDX_ROWS, sel_body, 0)

    def chunk_body(ci, carry):
        pbase = pair_base + ci * CP
        copies = [
            pltpu.async_copy(table_hbm.at[idx_v.at[ci * NGATHER + j]],
                             rows_v.at[pl.ds(j * 128, 128)], sem)
            for j in range(NGATHER)
        ]
        for c in copies:
            c.wait()

        def pair_body(p, c2):
            rb = p * TL
            for k in range(WD // 16):
                s = rows_v[rb, pl.ds(k * 16, 16)]
                for j in range(1, TL):
                    s = s + rows_v[rb + j, pl.ds(k * 16, 16)]
                acc_v[p, pl.ds(k * 16, 16)] = s
            return c2

        lax.fori_loop(0, CP, pair_body, 0)
        pltpu.sync_copy(acc_v, out_hbm.at[pl.ds(pbase, CP)])
        return carry

    lax.fori_loop(0, NCHUNK, chunk_body, 0)


@functools.cache
def _make_pool_sc():
    mesh = plsc.VectorSubcoreMesh(core_axis_name="c", subcore_axis_name="s")
    return pl.kernel(
        _pool_sc_body,
        mesh=mesh,
        out_type=jax.ShapeDtypeStruct((NPAIR, WD), jnp.float32),
        scratch_types=[
            pltpu.VMEM((IDX_ROWS, 128), jnp.int32),
            pltpu.VMEM((IDX_ROWS, 128), jnp.float32),
            pltpu.VMEM((IDX_CHUNK, WD), jnp.float32),
            pltpu.VMEM((CP, WD), jnp.float32),
            pltpu.SemaphoreType.DMA,
        ],
        compiler_params=pltpu.CompilerParams(use_tc_tiling_on_sc=False),
    )


BB = 8


def _dense_body(sums_ref, tmask_ref, gidx_ref, cmask_ref, graph_ref, cand_ref,
                Wn_ref, bn_ref, proxy_ref, W0_ref, b0_ref, W1_ref, b1_ref,
                Kw_ref, Qw_ref, Qb_ref, aW_ref, ab_ref, iKw_ref, iQw_ref,
                iQb_ref, out_ref):
    f32 = jnp.float32
    cnt = jnp.sum(tmask_ref[...], axis=2)
    pooled = sums_ref[...] / jnp.maximum(cnt, 1e-6)[..., None]
    hist = (pooled.reshape(BB * NH, WD) @ Wn_ref[...]
            + bn_ref[...])
    proxy = proxy_ref[...]
    W0 = W0_ref[...]
    W1 = W1_ref[...]
    b0 = b0_ref[...]
    b1 = b1_ref[...]
    Kw = Kw_ref[...]
    Qw = Qw_ref[...]
    Qb = Qb_ref[...]
    aW = aW_ref[...]
    ab = ab_ref[...]
    iKw = iKw_ref[...]
    iQw = iQw_ref[...]
    iQb = iQb_ref[...]
    cat_iota = lax.broadcasted_iota(jnp.int32, (NH, CATP), 1)

    for s in range(BB):
        h0 = jnp.concatenate([hist[s * NH:(s + 1) * NH], proxy], axis=0)
        A = graph_ref[s]
        t0 = A @ h0
        h1 = jax.nn.relu(t0 @ W0 + b0) + h0
        t1 = A @ h1
        h2 = t1 @ W1 + b1 + h1
        g = (h2 + h0)[:NH]

        K = g @ Kw
        cand = cand_ref[s]
        Q = cand @ Qw + Qb
        a = lax.dot_general(Q, K, (((1,), (1,)), ((), ()))) / SCALE

        idx = gidx_ref[s]
        oh = (idx[:, None] == cat_iota).astype(f32)
        seg_max = jnp.max(
            jnp.where(oh[None, :, :] > 0, a[:, :, None], -1e9), axis=1)
        max_g = lax.dot_general(seg_max, oh, (((1,), (1,)), ((), ())))
        exp_a = jnp.exp(a - max_g)
        seg_sum = exp_a @ oh
        denom = lax.dot_general(seg_sum, oh, (((1,), (1,)), ((), ())))
        alpha = exp_a / denom

        M = (alpha[:, None, :] * oh.T[None, :, :]).reshape(NN * CATP, NH)
        intra = M @ g
        intra = jax.nn.relu(intra @ aW + ab) + intra

        Kf = (intra @ iKw).reshape(NN, CATP, AD)
        Qf = cand @ iQw + iQb
        satt = jnp.sum(Kf * Qf[:, None, :], axis=2) / SCALE
        cm = cmask_ref[s]
        cm = jnp.where(lax.iota(jnp.int32, CATP) == CATP - 1, 1.0, cm)
        satt = jnp.where(cm[None, :] == 0, -1e9, satt)
        satt = satt - jnp.max(satt, axis=1, keepdims=True)
        e = jnp.exp(satt)
        al = e / jnp.sum(e, axis=1, keepdims=True)
        out_ref[s] = jnp.sum(
            al[:, :, None] * intra.reshape(NN, CATP, D), axis=1)


def _full(shape):
    return pl.BlockSpec(shape, lambda i: (0,) * len(shape))


_dense = pl.pallas_call(
    _dense_body,
    grid=(B // BB,),
    in_specs=[
        pl.BlockSpec((BB, NH, WD), lambda i: (i, 0, 0)),
        pl.BlockSpec((BB, NH, TL), lambda i: (i, 0, 0)),
        pl.BlockSpec((BB, NH), lambda i: (i, 0)),
        pl.BlockSpec((BB, CATP), lambda i: (i, 0)),
        pl.BlockSpec((BB, NODES, NODES), lambda i: (i, 0, 0)),
        pl.BlockSpec((BB, NN, D), lambda i: (i, 0, 0)),
        _full((WD, D)),
        _full((1, D)),
        _full((CAT, D)),
        _full((D, D)),
        _full((1, D)),
        _full((D, D)),
        _full((1, D)),
        _full((D, AD)),
        _full((D, AD)),
        _full((1, AD)),
        _full((D, D)),
        _full((1, D)),
        _full((D, AD)),
        _full((D, AD)),
        _full((1, AD)),
    ],
    out_specs=pl.BlockSpec((BB, NN, D), lambda i: (i, 0, 0)),
    out_shape=jax.ShapeDtypeStruct((B, NN, D), jnp.float32),
    compiler_params=pltpu.CompilerParams(
        dimension_semantics=("arbitrary",)),
)


def kernel(user_title_text, user_title_mask, user_title_entity,
           user_content_text, user_content_mask, user_content_entity,
           user_category, user_subCategory, user_history_mask,
           user_history_graph, user_history_category_mask,
           user_history_category_indices, user_embedding,
           candidate_news_representation, word_emb, W_news, b_news, proxy_emb,
           gcn_W0, gcn_b0, gcn_W1, gcn_b1, Kw, Qw, Qb, aff_W, aff_b, inter_Kw,
           inter_Qw, inter_Qb):
    idx2d = user_title_text.astype(jnp.int32).reshape(NW, IDX_ROWS, 128)
    mask2d = user_title_mask.reshape(NW, IDX_ROWS, 128)
    table_z = jnp.concatenate(
        [word_emb, jnp.zeros((1, WD), jnp.float32)], axis=0)
    sums = _make_pool_sc()(idx2d, mask2d, table_z)
    out = _dense(
        sums.reshape(B, NH, WD),
        user_title_mask,
        user_history_category_indices.astype(jnp.int32),
        user_history_category_mask,
        user_history_graph,
        candidate_news_representation,
        W_news,
        b_news.reshape(1, D),
        proxy_emb,
        gcn_W0,
        gcn_b0.reshape(1, D),
        gcn_W1,
        gcn_b1.reshape(1, D),
        Kw,
        Qw,
        Qb.reshape(1, AD),
        aff_W,
        aff_b.reshape(1, D),
        inter_Kw,
        inter_Qw,
        inter_Qb.reshape(1, AD),
    )
    return out


# speedup vs baseline: 5.7924x; 3.8116x over previous
import functools

import jax
import jax.numpy as jnp
from jax import lax
from jax.experimental import pallas as pl
from jax.experimental.pallas import tpu as pltpu
from jax.experimental.pallas import tpu_sc as plsc

B = 1024
NH = 50
NN = 5
D = 128
AD = 64
CAT = 18
CATP = 19
TL = 20
V = 30000
WD = 64
NODES = NH + CAT
SCALE = 8.0


NC = 2
NS = 16
NW = NC * NS
NPAIR = B * NH
PAIRS_PER_W = NPAIR // NW
CP = 32
NCHUNK = PAIRS_PER_W // CP
IDX_CHUNK = CP * TL
NGATHER = IDX_CHUNK // 128

IDX_ROWS = PAIRS_PER_W * TL // 128

_UNPACK_PERM = []
for _kk in range(WD // 32):
    _UNPACK_PERM += [32 * _kk + 2 * _j for _j in range(16)]
    _UNPACK_PERM += [32 * _kk + 2 * _j + 1 for _j in range(16)]


def _pool_sc_body(idx_hbm, mask_hbm, table_hbm, out_hbm, table_sh, idx_v,
                  mask_v, rows_v, acc_v, sem):
    sid = lax.axis_index("s")
    wid = sid * NC + lax.axis_index("c")
    pair_base = wid * PAIRS_PER_W

    @pl.when(sid == 0)
    def _load_table():
        pltpu.sync_copy(table_hbm, table_sh)

    pltpu.sync_copy(idx_hbm.at[wid], idx_v)

    def mask_chunk(ci):
        pltpu.sync_copy(
            mask_hbm.at[pl.ds(wid * PAIRS_PER_W * TL + ci * IDX_CHUNK,
                              IDX_CHUNK)], mask_v)
        for j in range(NGATHER):
            for i in range(8):
                m = mask_v[pl.ds((j * 8 + i) * 16, 16)]
                iv = idx_v[ci * NGATHER + j, pl.ds(i * 16, 16)]
                idx_v[ci * NGATHER + j, pl.ds(i * 16, 16)] = (
                    jnp.where(m > 0.0, iv, V))

    def sel_body(ci, carry):
        mask_chunk(ci)
        return carry

    lax.fori_loop(0, NCHUNK, sel_body, 0)
    plsc.subcore_barrier()

    def chunk_body(ci, carry):
        pbase = pair_base + ci * CP
        copies = [
            pltpu.async_copy(table_sh.at[idx_v.at[ci * NGATHER + j]],
                             rows_v.at[pl.ds(j * 128, 128)], sem)
            for j in range(NGATHER)
        ]
        for c in copies:
            c.wait()

        def pair_body(p, c2):
            rb = p * TL
            himask = jnp.int32(-65536)
            for kk in range(WD // 32):
                def unpack2(w):
                    lo = lax.bitcast_convert_type(
                        lax.shift_left(w, 16), jnp.float32)
                    hi = lax.bitcast_convert_type(
                        lax.bitwise_and(w, himask), jnp.float32)
                    return lo, hi

                sa, sb = unpack2(rows_v[rb, pl.ds(kk * 16, 16)])
                for j in range(1, TL):
                    a, b = unpack2(rows_v[rb + j, pl.ds(kk * 16, 16)])
                    sa = sa + a
                    sb = sb + b
                acc_v[p, pl.ds(kk * 32, 16)] = sa
                acc_v[p, pl.ds(kk * 32 + 16, 16)] = sb
            return c2

        lax.fori_loop(0, CP, pair_body, 0)
        pltpu.sync_copy(acc_v, out_hbm.at[pl.ds(pbase, CP)])
        return carry

    lax.fori_loop(0, NCHUNK, chunk_body, 0)


@functools.cache
def _make_pool_sc():
    mesh = plsc.VectorSubcoreMesh(core_axis_name="c", subcore_axis_name="s")
    return pl.kernel(
        _pool_sc_body,
        mesh=mesh,
        out_type=jax.ShapeDtypeStruct((NPAIR, WD), jnp.float32),
        scratch_types=[
            pltpu.VMEM_SHARED((V + 1, WD // 2), jnp.int32),
            pltpu.VMEM((IDX_ROWS, 128), jnp.int32),
            pltpu.VMEM((IDX_CHUNK,), jnp.float32),
            pltpu.VMEM((IDX_CHUNK, WD // 2), jnp.int32),
            pltpu.VMEM((CP, WD), jnp.float32),
            pltpu.SemaphoreType.DMA,
        ],
        compiler_params=pltpu.CompilerParams(use_tc_tiling_on_sc=False),
    )


BB = 8


def _dense_body(sums_ref, tmask_ref, gidx_ref, cmask_ref, graph_ref, cand_ref,
                Wn_ref, bn_ref, proxy_ref, W0_ref, b0_ref, W1_ref, b1_ref,
                Kw_ref, Qw_ref, Qb_ref, aW_ref, ab_ref, iKw_ref, iQw_ref,
                iQb_ref, out_ref):
    f32 = jnp.float32
    cnt = jnp.sum(tmask_ref[...], axis=2)
    pooled = sums_ref[...] / jnp.maximum(cnt, 1e-6)[..., None]
    hist = (pooled.reshape(BB * NH, WD) @ Wn_ref[...]
            + bn_ref[...])
    proxy = proxy_ref[...]
    W0 = W0_ref[...]
    W1 = W1_ref[...]
    b0 = b0_ref[...]
    b1 = b1_ref[...]
    Kw = Kw_ref[...]
    Qw = Qw_ref[...]
    Qb = Qb_ref[...]
    aW = aW_ref[...]
    ab = ab_ref[...]
    iKw = iKw_ref[...]
    iQw = iQw_ref[...]
    iQb = iQb_ref[...]
    cat_iota = lax.broadcasted_iota(jnp.int32, (NH, CATP), 1)

    for s in range(BB):
        h0 = jnp.concatenate([hist[s * NH:(s + 1) * NH], proxy], axis=0)
        A = graph_ref[s]
        t0 = A @ h0
        h1 = jax.nn.relu(t0 @ W0 + b0) + h0
        t1 = A @ h1
        h2 = t1 @ W1 + b1 + h1
        g = (h2 + h0)[:NH]

        K = g @ Kw
        cand = cand_ref[s]
        Q = cand @ Qw + Qb
        a = lax.dot_general(Q, K, (((1,), (1,)), ((), ()))) / SCALE

        idx = gidx_ref[s]
        oh = (idx[:, None] == cat_iota).astype(f32)
        seg_max = jnp.max(
            jnp.where(oh[None, :, :] > 0, a[:, :, None], -1e9), axis=1)
        max_g = lax.dot_general(seg_max, oh, (((1,), (1,)), ((), ())))
        exp_a = jnp.exp(a - max_g)
        seg_sum = exp_a @ oh
        denom = lax.dot_general(seg_sum, oh, (((1,), (1,)), ((), ())))
        alpha = exp_a / denom

        M = (alpha[:, None, :] * oh.T[None, :, :]).reshape(NN * CATP, NH)
        intra = M @ g
        intra = jax.nn.relu(intra @ aW + ab) + intra

        Kf = (intra @ iKw).reshape(NN, CATP, AD)
        Qf = cand @ iQw + iQb
        satt = jnp.sum(Kf * Qf[:, None, :], axis=2) / SCALE
        cm = cmask_ref[s]
        cm = jnp.where(lax.iota(jnp.int32, CATP) == CATP - 1, 1.0, cm)
        satt = jnp.where(cm[None, :] == 0, -1e9, satt)
        satt = satt - jnp.max(satt, axis=1, keepdims=True)
        e = jnp.exp(satt)
        al = e / jnp.sum(e, axis=1, keepdims=True)
        out_ref[s] = jnp.sum(
            al[:, :, None] * intra.reshape(NN, CATP, D), axis=1)


def _full(shape):
    return pl.BlockSpec(shape, lambda i: (0,) * len(shape))


_dense = pl.pallas_call(
    _dense_body,
    grid=(B // BB,),
    in_specs=[
        pl.BlockSpec((BB, NH, WD), lambda i: (i, 0, 0)),
        pl.BlockSpec((BB, NH, TL), lambda i: (i, 0, 0)),
        pl.BlockSpec((BB, NH), lambda i: (i, 0)),
        pl.BlockSpec((BB, CATP), lambda i: (i, 0)),
        pl.BlockSpec((BB, NODES, NODES), lambda i: (i, 0, 0)),
        pl.BlockSpec((BB, NN, D), lambda i: (i, 0, 0)),
        _full((WD, D)),
        _full((1, D)),
        _full((CAT, D)),
        _full((D, D)),
        _full((1, D)),
        _full((D, D)),
        _full((1, D)),
        _full((D, AD)),
        _full((D, AD)),
        _full((1, AD)),
        _full((D, D)),
        _full((1, D)),
        _full((D, AD)),
        _full((D, AD)),
        _full((1, AD)),
    ],
    out_specs=pl.BlockSpec((BB, NN, D), lambda i: (i, 0, 0)),
    out_shape=jax.ShapeDtypeStruct((B, NN, D), jnp.float32),
    compiler_params=pltpu.CompilerParams(
        dimension_semantics=("arbitrary",)),
)


def kernel(user_title_text, user_title_mask, user_title_entity,
           user_content_text, user_content_mask, user_content_entity,
           user_category, user_subCategory, user_history_mask,
           user_history_graph, user_history_category_mask,
           user_history_category_indices, user_embedding,
           candidate_news_representation, word_emb, W_news, b_news, proxy_emb,
           gcn_W0, gcn_b0, gcn_W1, gcn_b1, Kw, Qw, Qb, aff_W, aff_b, inter_Kw,
           inter_Qw, inter_Qb):
    idx2d = user_title_text.astype(jnp.int32).reshape(NW, IDX_ROWS, 128)
    mask1d = user_title_mask.reshape(-1)
    table_bf = jnp.concatenate(
        [word_emb.astype(jnp.bfloat16),
         jnp.zeros((1, WD), jnp.bfloat16)], axis=0)
    table_z = lax.bitcast_convert_type(
        table_bf.reshape(V + 1, WD // 2, 2), jnp.int32)
    sums = _make_pool_sc()(idx2d, mask1d, table_z)
    out = _dense(
        sums.reshape(B, NH, WD),
        user_title_mask,
        user_history_category_indices.astype(jnp.int32),
        user_history_category_mask,
        user_history_graph,
        candidate_news_representation,
        W_news[jnp.array(_UNPACK_PERM)],
        b_news.reshape(1, D),
        proxy_emb,
        gcn_W0,
        gcn_b0.reshape(1, D),
        gcn_W1,
        gcn_b1.reshape(1, D),
        Kw,
        Qw,
        Qb.reshape(1, AD),
        aff_W,
        aff_b.reshape(1, D),
        inter_Kw,
        inter_Qw,
        inter_Qb.reshape(1, AD),
    )
    return out


# speedup vs baseline: 11.4601x; 1.9785x over previous
import functools

import jax
import jax.numpy as jnp
from jax import lax
from jax.experimental import pallas as pl
from jax.experimental.pallas import tpu as pltpu
from jax.experimental.pallas import tpu_sc as plsc

B = 1024
NH = 50
NN = 5
D = 128
AD = 64
CAT = 18
CATP = 19
TL = 20
V = 30000
WD = 64
NODES = NH + CAT
SCALE = 8.0


NC = 2
NS = 16
NW = NC * NS
NPAIR = B * NH
PAIRS_PER_W = NPAIR // NW
CP = 32
NCHUNK = PAIRS_PER_W // CP
IDX_CHUNK = CP * TL
NGATHER = IDX_CHUNK // 128

IDX_ROWS = PAIRS_PER_W * TL // 128

_UNPACK_PERM = []
for _kk in range(WD // 32):
    _UNPACK_PERM += [32 * _kk + 2 * _j for _j in range(16)]
    _UNPACK_PERM += [32 * _kk + 2 * _j + 1 for _j in range(16)]


def _pool_sc_body(idx_hbm, mask_hbm, table_hbm, out_hbm, table_sh, idx_v,
                  mask_v, rows_v, acc_v, sem):
    sid = lax.axis_index("s")
    wid = sid * NC + lax.axis_index("c")
    pair_base = wid * PAIRS_PER_W

    @pl.when(sid == 0)
    def _load_table():
        pltpu.sync_copy(table_hbm, table_sh)

    pltpu.sync_copy(idx_hbm.at[wid], idx_v)

    def mask_chunk(ci):
        pltpu.sync_copy(
            mask_hbm.at[pl.ds(wid * PAIRS_PER_W * TL + ci * IDX_CHUNK,
                              IDX_CHUNK)], mask_v)
        for j in range(NGATHER):
            for i in range(8):
                m = mask_v[pl.ds((j * 8 + i) * 16, 16)]
                iv = idx_v[ci * NGATHER + j, pl.ds(i * 16, 16)]
                idx_v[ci * NGATHER + j, pl.ds(i * 16, 16)] = (
                    jnp.where(m > 0.0, iv, V))

    def sel_body(ci, carry):
        mask_chunk(ci)
        return carry

    lax.fori_loop(0, NCHUNK, sel_body, 0)
    plsc.subcore_barrier()

    def chunk_body(ci, carry):
        pbase = pair_base + ci * CP
        copies = [
            pltpu.async_copy(table_sh.at[idx_v.at[ci * NGATHER + j]],
                             rows_v.at[pl.ds(j * 128, 128)], sem)
            for j in range(NGATHER)
        ]
        for c in copies:
            c.wait()

        def pair_body(p, c2):
            rb = p * TL
            himask = jnp.int32(-65536)
            for kk in range(WD // 32):
                def unpack2(w):
                    lo = lax.bitcast_convert_type(
                        lax.shift_left(w, 16), jnp.float32)
                    hi = lax.bitcast_convert_type(
                        lax.bitwise_and(w, himask), jnp.float32)
                    return lo, hi

                sa, sb = unpack2(rows_v[rb, pl.ds(kk * 16, 16)])
                for j in range(1, TL):
                    a, b = unpack2(rows_v[rb + j, pl.ds(kk * 16, 16)])
                    sa = sa + a
                    sb = sb + b
                acc_v[p, pl.ds(kk * 32, 16)] = sa
                acc_v[p, pl.ds(kk * 32 + 16, 16)] = sb
            return c2

        lax.fori_loop(0, CP, pair_body, 0)
        pltpu.sync_copy(acc_v, out_hbm.at[pl.ds(pbase, CP)])
        return carry

    lax.fori_loop(0, NCHUNK, chunk_body, 0)


@functools.cache
def _make_pool_sc():
    mesh = plsc.VectorSubcoreMesh(core_axis_name="c", subcore_axis_name="s")
    return pl.kernel(
        _pool_sc_body,
        mesh=mesh,
        out_type=jax.ShapeDtypeStruct((NPAIR, WD), jnp.float32),
        scratch_types=[
            pltpu.VMEM_SHARED((V + 1, WD // 2), jnp.int32),
            pltpu.VMEM((IDX_ROWS, 128), jnp.int32),
            pltpu.VMEM((IDX_CHUNK,), jnp.float32),
            pltpu.VMEM((IDX_CHUNK, WD // 2), jnp.int32),
            pltpu.VMEM((CP, WD), jnp.float32),
            pltpu.SemaphoreType.DMA,
        ],
        compiler_params=pltpu.CompilerParams(use_tc_tiling_on_sc=False),
    )


BB = 8
NHP = 56
NP = 80
CATPP = 24


def _dense_body(sums_ref, tmask_ref, gidx_ref, cmask_ref, graph_ref, cand_ref,
                Wn_ref, bn_ref, proxy_ref, W0_ref, b0_ref, W1_ref, b1_ref,
                Kw_ref, Qw_ref, Qb_ref, aW_ref, ab_ref, iKw_ref, iQw_ref,
                iQb_ref, out_ref):
    f32 = jnp.float32
    cnt = jnp.sum(tmask_ref[...], axis=1, keepdims=True)
    pooled = sums_ref[...] / jnp.maximum(cnt, 1e-6)
    hist = pooled @ Wn_ref[...] + bn_ref[...]
    proxy = proxy_ref[...]
    W0 = W0_ref[...]
    W1 = W1_ref[...]
    b0 = b0_ref[...]
    b1 = b1_ref[...]
    cand = cand_ref[...]

    h0s = [jnp.concatenate([hist[s * NHP:(s + 1) * NHP], proxy], axis=0)
           for s in range(BB)]
    H0 = jnp.concatenate(h0s, axis=0)
    T0 = jnp.concatenate(
        [graph_ref[s] @ h0s[s] for s in range(BB)], axis=0)
    H1 = jax.nn.relu(T0 @ W0 + b0) + H0
    T1 = jnp.concatenate(
        [graph_ref[s] @ H1[s * NP:(s + 1) * NP] for s in range(BB)], axis=0)
    G = T1 @ W1 + b1 + H1 + H0

    K = G @ Kw_ref[...]
    Q = cand @ Qw_ref[...] + Qb_ref[...]
    cat_iota = lax.broadcasted_iota(jnp.int32, (CATPP, NH), 0)

    intras = []
    atts = []
    for s in range(BB):
        K_s = K[s * NP:s * NP + NH]
        Q_s = Q[s * NN:(s + 1) * NN]
        a = lax.dot_general(Q_s, K_s, (((1,), (1,)), ((), ()))) / SCALE
        idx = gidx_ref[s]
        ohT = (cat_iota == idx[None, :]).astype(f32)
        seg_max = jnp.max(
            jnp.where(ohT[None, :, :] > 0, a[:, None, :], -1e9), axis=2)
        max_g = seg_max @ ohT
        exp_a = jnp.exp(a - max_g)
        seg_sum = lax.dot_general(exp_a, ohT, (((1,), (1,)), ((), ())))
        denom = seg_sum @ ohT
        alpha = exp_a / denom
        M = jnp.concatenate(
            [ohT * alpha[n:n + 1, :] for n in range(NN)], axis=0)
        intras.append(M @ G[s * NP:s * NP + NH])

    INTRA = jnp.concatenate(intras, axis=0)
    INTRA = jax.nn.relu(INTRA @ aW_ref[...] + ab_ref[...]) + INTRA
    KF = INTRA @ iKw_ref[...]
    QF = cand @ iQw_ref[...] + iQb_ref[...]
    SNC = NN * CATPP

    for s in range(BB):
        Kf3 = KF[s * SNC:(s + 1) * SNC].reshape(NN, CATPP, AD)
        Qf_s = QF[s * NN:(s + 1) * NN]
        satt = jnp.sum(Kf3 * Qf_s[:, None, :], axis=2) / SCALE
        cm = cmask_ref[s]
        satt = jnp.where(cm[None, :] == 0, -1e9, satt)
        satt = satt - jnp.max(satt, axis=1, keepdims=True)
        e = jnp.exp(satt)
        al = e / jnp.sum(e, axis=1, keepdims=True)
        intra3 = INTRA[s * SNC:(s + 1) * SNC].reshape(NN, CATPP, D)
        out_ref[pl.ds(s * NN, NN), :] = jnp.sum(
            al[:, :, None] * intra3, axis=1)


def _full(shape):
    return pl.BlockSpec(shape, lambda i: (0,) * len(shape))


_dense = pl.pallas_call(
    _dense_body,
    grid=(B // BB,),
    in_specs=[
        pl.BlockSpec((BB * NHP, WD), lambda i: (i, 0)),
        pl.BlockSpec((BB * NHP, TL), lambda i: (i, 0)),
        pl.BlockSpec((BB, NH), lambda i: (i, 0)),
        pl.BlockSpec((BB, CATPP), lambda i: (i, 0)),
        pl.BlockSpec((BB, NP, NP), lambda i: (i, 0, 0)),
        pl.BlockSpec((BB * NN, D), lambda i: (i, 0)),
        _full((WD, D)),
        _full((1, D)),
        _full((CATPP, D)),
        _full((D, D)),
        _full((1, D)),
        _full((D, D)),
        _full((1, D)),
        _full((D, AD)),
        _full((D, AD)),
        _full((1, AD)),
        _full((D, D)),
        _full((1, D)),
        _full((D, AD)),
        _full((D, AD)),
        _full((1, AD)),
    ],
    out_specs=pl.BlockSpec((BB * NN, D), lambda i: (i, 0)),
    out_shape=jax.ShapeDtypeStruct((B * NN, D), jnp.float32),
    compiler_params=pltpu.CompilerParams(
        dimension_semantics=("arbitrary",)),
)


def kernel(user_title_text, user_title_mask, user_title_entity,
           user_content_text, user_content_mask, user_content_entity,
           user_category, user_subCategory, user_history_mask,
           user_history_graph, user_history_category_mask,
           user_history_category_indices, user_embedding,
           candidate_news_representation, word_emb, W_news, b_news, proxy_emb,
           gcn_W0, gcn_b0, gcn_W1, gcn_b1, Kw, Qw, Qb, aff_W, aff_b, inter_Kw,
           inter_Qw, inter_Qb):
    idx2d = user_title_text.astype(jnp.int32).reshape(NW, IDX_ROWS, 128)
    mask1d = user_title_mask.reshape(-1)
    table_bf = jnp.concatenate(
        [word_emb.astype(jnp.bfloat16),
         jnp.zeros((1, WD), jnp.bfloat16)], axis=0)
    table_z = lax.bitcast_convert_type(
        table_bf.reshape(V + 1, WD // 2, 2), jnp.int32)
    sums = _make_pool_sc()(idx2d, mask1d, table_z)

    sums_p = jnp.pad(sums.reshape(B, NH, WD),
                     ((0, 0), (0, NHP - NH), (0, 0))).reshape(B * NHP, WD)
    tmask_p = jnp.pad(user_title_mask,
                      ((0, 0), (0, NHP - NH), (0, 0))).reshape(B * NHP, TL)
    Ag = user_history_graph
    zc = jnp.zeros((B, NH, NHP - NH), jnp.float32)
    zc2 = jnp.zeros((B, CAT, NHP - NH), jnp.float32)
    top = jnp.concatenate(
        [Ag[:, :NH, :NH], zc, Ag[:, :NH, NH:], zc], axis=2)
    bot = jnp.concatenate(
        [Ag[:, NH:, :NH], zc2, Ag[:, NH:, NH:], zc2], axis=2)
    graph_p = jnp.concatenate(
        [top, jnp.zeros((B, NHP - NH, NP), jnp.float32), bot,
         jnp.zeros((B, NP - NHP - CAT, NP), jnp.float32)], axis=1)
    proxy_p = jnp.pad(proxy_emb, ((0, CATPP - CAT + 1), (0, 0)))[:CATPP]
    cmask_p = jnp.pad(
        user_history_category_mask.at[:, -1].set(1.0),
        ((0, 0), (0, CATPP - CATP)))
    cand2 = candidate_news_representation.reshape(B * NN, D)

    out = _dense(
        sums_p,
        tmask_p,
        user_history_category_indices.astype(jnp.int32),
        cmask_p,
        graph_p,
        cand2,
        W_news[jnp.array(_UNPACK_PERM)],
        b_news.reshape(1, D),
        proxy_p,
        gcn_W0,
        gcn_b0.reshape(1, D),
        gcn_W1,
        gcn_b1.reshape(1, D),
        Kw,
        Qw,
        Qb.reshape(1, AD),
        aff_W,
        aff_b.reshape(1, D),
        inter_Kw,
        inter_Qw,
        inter_Qb.reshape(1, AD),
    )
    return out.reshape(B, NN, D)


# speedup vs baseline: 14.4402x; 1.2600x over previous
import functools

import jax
import jax.numpy as jnp
from jax import lax
from jax.experimental import pallas as pl
from jax.experimental.pallas import tpu as pltpu
from jax.experimental.pallas import tpu_sc as plsc

B = 1024
NH = 50
NN = 5
D = 128
AD = 64
CAT = 18
CATP = 19
TL = 20
V = 30000
WD = 64
NODES = NH + CAT
SCALE = 8.0


NC = 2
NS = 16
NW = NC * NS
NPAIR = B * NH
PAIRS_PER_W = NPAIR // NW
CP = 32
NCHUNK = PAIRS_PER_W // CP
IDX_CHUNK = CP * TL
NGATHER = IDX_CHUNK // 128

IDX_ROWS = PAIRS_PER_W * TL // 128

_UNPACK_PERM = []
for _kk in range(WD // 32):
    _UNPACK_PERM += [32 * _kk + 2 * _j for _j in range(16)]
    _UNPACK_PERM += [32 * _kk + 2 * _j + 1 for _j in range(16)]


def _pool_sc_body(idx_hbm, mask_hbm, table_hbm, out_hbm, table_sh, idx_v,
                  mask_v, rows_v, acc_v, sem):
    sid = lax.axis_index("s")
    wid = sid * NC + lax.axis_index("c")
    pair_base = wid * PAIRS_PER_W

    @pl.when(sid == 0)
    def _load_table():
        pltpu.sync_copy(table_hbm, table_sh)

    pltpu.sync_copy(idx_hbm.at[wid], idx_v)

    def mask_chunk(ci):
        pltpu.sync_copy(
            mask_hbm.at[pl.ds(wid * PAIRS_PER_W * TL + ci * IDX_CHUNK,
                              IDX_CHUNK)], mask_v)
        for j in range(NGATHER):
            for i in range(8):
                m = mask_v[pl.ds((j * 8 + i) * 16, 16)]
                iv = idx_v[ci * NGATHER + j, pl.ds(i * 16, 16)]
                idx_v[ci * NGATHER + j, pl.ds(i * 16, 16)] = (
                    jnp.where(m > 0.0, iv, V))

    def sel_body(ci, carry):
        mask_chunk(ci)
        return carry

    lax.fori_loop(0, NCHUNK, sel_body, 0)
    plsc.subcore_barrier()

    def chunk_body(ci, carry):
        pbase = pair_base + ci * CP
        copies = [
            pltpu.async_copy(table_sh.at[idx_v.at[ci * NGATHER + j]],
                             rows_v.at[pl.ds(j * 128, 128)], sem)
            for j in range(NGATHER)
        ]
        for c in copies:
            c.wait()

        def pair_body(p, c2):
            rb = p * TL
            himask = jnp.int32(-65536)
            for kk in range(WD // 32):
                def unpack2(w):
                    lo = lax.bitcast_convert_type(
                        lax.shift_left(w, 16), jnp.float32)
                    hi = lax.bitcast_convert_type(
                        lax.bitwise_and(w, himask), jnp.float32)
                    return lo, hi

                sa, sb = unpack2(rows_v[rb, pl.ds(kk * 16, 16)])
                for j in range(1, TL):
                    a, b = unpack2(rows_v[rb + j, pl.ds(kk * 16, 16)])
                    sa = sa + a
                    sb = sb + b
                acc_v[p, pl.ds(kk * 32, 16)] = sa
                acc_v[p, pl.ds(kk * 32 + 16, 16)] = sb
            return c2

        lax.fori_loop(0, CP, pair_body, 0)
        pltpu.sync_copy(acc_v, out_hbm.at[pl.ds(pbase, CP)])
        return carry

    lax.fori_loop(0, NCHUNK, chunk_body, 0)


@functools.cache
def _make_pool_sc():
    mesh = plsc.VectorSubcoreMesh(core_axis_name="c", subcore_axis_name="s")
    return pl.kernel(
        _pool_sc_body,
        mesh=mesh,
        out_type=jax.ShapeDtypeStruct((NPAIR, WD), jnp.float32),
        scratch_types=[
            pltpu.VMEM_SHARED((V + 1, WD // 2), jnp.int32),
            pltpu.VMEM((IDX_ROWS, 128), jnp.int32),
            pltpu.VMEM((IDX_CHUNK,), jnp.float32),
            pltpu.VMEM((IDX_CHUNK, WD // 2), jnp.int32),
            pltpu.VMEM((CP, WD), jnp.float32),
            pltpu.SemaphoreType.DMA,
        ],
        compiler_params=pltpu.CompilerParams(use_tc_tiling_on_sc=False),
    )


BB = 8
NHP = 56
NP = 80
CATPP = 24


def _dense_body(sums_ref, tmask_ref, gidx_ref, cmask_ref, graph_ref, cand_ref,
                Wn_ref, bn_ref, proxy_ref, W0_ref, b0_ref, W1_ref, b1_ref,
                Kw_ref, Qw_ref, Qb_ref, aW_ref, ab_ref, iKw_ref, iQw_ref,
                iQb_ref, out_ref):
    f32 = jnp.float32
    bf = jnp.bfloat16

    def mm(x, w):
        return lax.dot_general(x.astype(bf), w, (((1,), (0,)), ((), ())),
                               preferred_element_type=f32)

    cnt = jnp.sum(tmask_ref[...], axis=1, keepdims=True)
    pooled = sums_ref[...] / jnp.maximum(cnt, 1e-6)
    hist = mm(pooled, Wn_ref[...]) + bn_ref[...]
    proxy = proxy_ref[...]
    W0 = W0_ref[...]
    W1 = W1_ref[...]
    b0 = b0_ref[...]
    b1 = b1_ref[...]
    cand = cand_ref[...]

    h0s = [jnp.concatenate([hist[s * NHP:(s + 1) * NHP], proxy], axis=0)
           for s in range(BB)]
    H0 = jnp.concatenate(h0s, axis=0)
    T0 = jnp.concatenate(
        [lax.dot_general(graph_ref[s], h0s[s].astype(bf),
                         (((1,), (0,)), ((), ())), preferred_element_type=f32)
         for s in range(BB)], axis=0)
    H1 = jax.nn.relu(mm(T0, W0) + b0) + H0
    T1 = jnp.concatenate(
        [lax.dot_general(graph_ref[s], H1[s * NP:(s + 1) * NP].astype(bf),
                         (((1,), (0,)), ((), ())), preferred_element_type=f32)
         for s in range(BB)], axis=0)
    G = mm(T1, W1) + b1 + H1 + H0

    K = mm(G, Kw_ref[...]).astype(bf)
    Q = (mm(cand, Qw_ref[...]) + Qb_ref[...]).astype(bf)
    cat_iota = lax.broadcasted_iota(jnp.int32, (CATPP, NH), 0)

    a_list = []
    oh_list = []
    for s in range(BB):
        K_s = K[s * NP:s * NP + NH]
        Q_s = Q[s * NN:(s + 1) * NN]
        a_list.append(
            lax.dot_general(Q_s, K_s, (((1,), (1,)), ((), ())),
                            preferred_element_type=f32) / SCALE)
        oh_list.append(
            (cat_iota == gidx_ref[s][None, :]).astype(f32))
    A3 = jnp.stack(a_list)
    OH3 = jnp.stack(oh_list)
    SEGMAX = jnp.max(
        jnp.where(OH3[:, None, :, :] > 0, A3[:, :, None, :], -1e9), axis=3)
    MG3 = jnp.stack([SEGMAX[s] @ OH3[s] for s in range(BB)])
    EXPA = jnp.exp(A3 - MG3)
    DEN3 = jnp.stack(
        [lax.dot_general(EXPA[s], OH3[s], (((1,), (1,)), ((), ()))) @ OH3[s]
         for s in range(BB)])
    AL3 = EXPA / DEN3

    intras = []
    for s in range(BB):
        M = jnp.concatenate(
            [OH3[s] * AL3[s, n:n + 1, :] for n in range(NN)], axis=0)
        intras.append(
            lax.dot_general(M.astype(bf),
                            G[s * NP:s * NP + NH].astype(bf),
                            (((1,), (0,)), ((), ())),
                            preferred_element_type=f32))

    INTRA = jnp.concatenate(intras, axis=0)
    INTRA = jax.nn.relu(mm(INTRA, aW_ref[...]) + ab_ref[...]) + INTRA
    KF = mm(INTRA, iKw_ref[...])
    QF = mm(cand, iQw_ref[...]) + iQb_ref[...]

    KF3 = KF.reshape(BB * NN, CATPP, AD)
    satt = jnp.sum(KF3 * QF[:, None, :], axis=2) / SCALE
    satt = jnp.where(cmask_ref[...] == 0, -1e9, satt)
    satt = satt - jnp.max(satt, axis=1, keepdims=True)
    e = jnp.exp(satt)
    al = e / jnp.sum(e, axis=1, keepdims=True)
    out_ref[...] = jnp.sum(
        INTRA.reshape(BB * NN, CATPP, D) * al[:, :, None], axis=1)


def _full(shape):
    return pl.BlockSpec(shape, lambda i: (0,) * len(shape))


_dense = pl.pallas_call(
    _dense_body,
    grid=(B // BB,),
    in_specs=[
        pl.BlockSpec((BB * NHP, WD), lambda i: (i, 0)),
        pl.BlockSpec((BB * NHP, TL), lambda i: (i, 0)),
        pl.BlockSpec((BB, NH), lambda i: (i, 0)),
        pl.BlockSpec((BB * NN, CATPP), lambda i: (i, 0)),
        pl.BlockSpec((BB, NP, NP), lambda i: (i, 0, 0)),
        pl.BlockSpec((BB * NN, D), lambda i: (i, 0)),
        _full((WD, D)),
        _full((1, D)),
        _full((CATPP, D)),
        _full((D, D)),
        _full((1, D)),
        _full((D, D)),
        _full((1, D)),
        _full((D, AD)),
        _full((D, AD)),
        _full((1, AD)),
        _full((D, D)),
        _full((1, D)),
        _full((D, AD)),
        _full((D, AD)),
        _full((1, AD)),
    ],
    out_specs=pl.BlockSpec((BB * NN, D), lambda i: (i, 0)),
    out_shape=jax.ShapeDtypeStruct((B * NN, D), jnp.float32),
    compiler_params=pltpu.CompilerParams(
        dimension_semantics=("arbitrary",)),
)


def kernel(user_title_text, user_title_mask, user_title_entity,
           user_content_text, user_content_mask, user_content_entity,
           user_category, user_subCategory, user_history_mask,
           user_history_graph, user_history_category_mask,
           user_history_category_indices, user_embedding,
           candidate_news_representation, word_emb, W_news, b_news, proxy_emb,
           gcn_W0, gcn_b0, gcn_W1, gcn_b1, Kw, Qw, Qb, aff_W, aff_b, inter_Kw,
           inter_Qw, inter_Qb):
    idx2d = user_title_text.astype(jnp.int32).reshape(NW, IDX_ROWS, 128)
    mask1d = user_title_mask.reshape(-1)
    table_bf = jnp.concatenate(
        [word_emb.astype(jnp.bfloat16),
         jnp.zeros((1, WD), jnp.bfloat16)], axis=0)
    table_z = lax.bitcast_convert_type(
        table_bf.reshape(V + 1, WD // 2, 2), jnp.int32)
    sums = _make_pool_sc()(idx2d, mask1d, table_z)

    sums_p = jnp.pad(sums.reshape(B, NH, WD),
                     ((0, 0), (0, NHP - NH), (0, 0))).reshape(B * NHP, WD)
    tmask_p = jnp.pad(user_title_mask,
                      ((0, 0), (0, NHP - NH), (0, 0))).reshape(B * NHP, TL)
    Ag = user_history_graph
    zc = jnp.zeros((B, NH, NHP - NH), jnp.float32)
    zc2 = jnp.zeros((B, CAT, NHP - NH), jnp.float32)
    top = jnp.concatenate(
        [Ag[:, :NH, :NH], zc, Ag[:, :NH, NH:], zc], axis=2)
    bot = jnp.concatenate(
        [Ag[:, NH:, :NH], zc2, Ag[:, NH:, NH:], zc2], axis=2)
    graph_p = jnp.concatenate(
        [top, jnp.zeros((B, NHP - NH, NP), jnp.float32), bot,
         jnp.zeros((B, NP - NHP - CAT, NP), jnp.float32)], axis=1)
    proxy_p = jnp.pad(proxy_emb, ((0, CATPP - CAT + 1), (0, 0)))[:CATPP]
    cmask_p = jnp.repeat(
        jnp.pad(user_history_category_mask.at[:, -1].set(1.0),
                ((0, 0), (0, CATPP - CATP))), NN, axis=0)
    cand2 = candidate_news_representation.reshape(B * NN, D)

    bf = jnp.bfloat16
    out = _dense(
        sums_p,
        tmask_p,
        user_history_category_indices.astype(jnp.int32),
        cmask_p,
        graph_p.astype(bf),
        cand2,
        W_news[jnp.array(_UNPACK_PERM)].astype(bf),
        b_news.reshape(1, D),
        proxy_p,
        gcn_W0.astype(bf),
        gcn_b0.reshape(1, D),
        gcn_W1.astype(bf),
        gcn_b1.reshape(1, D),
        Kw.astype(bf),
        Qw.astype(bf),
        Qb.reshape(1, AD),
        aff_W.astype(bf),
        aff_b.reshape(1, D),
        inter_Kw.astype(bf),
        inter_Qw.astype(bf),
        inter_Qb.reshape(1, AD),
    )
    return out.reshape(B, NN, D)


# speedup vs baseline: 16.5550x; 1.1465x over previous
import functools

import jax
import jax.numpy as jnp
from jax import lax
from jax.experimental import pallas as pl
from jax.experimental.pallas import tpu as pltpu
from jax.experimental.pallas import tpu_sc as plsc

B = 1024
NH = 50
NN = 5
D = 128
AD = 64
CAT = 18
CATP = 19
TL = 20
V = 30000
WD = 64
NODES = NH + CAT
SCALE = 8.0


NC = 2
NS = 16
NW = NC * NS
NPAIR = B * NH
PAIRS_PER_W = NPAIR // NW
CP = 32
NCHUNK = PAIRS_PER_W // CP
IDX_CHUNK = CP * TL
NGATHER = IDX_CHUNK // 128

IDX_ROWS = PAIRS_PER_W * TL // 128

_UNPACK_PERM = []
for _kk in range(WD // 32):
    _UNPACK_PERM += [32 * _kk + 2 * _j for _j in range(16)]
    _UNPACK_PERM += [32 * _kk + 2 * _j + 1 for _j in range(16)]


def _pool_sc_body(idx_hbm, mask_hbm, table_hbm, out_hbm, table_sh, idx_v,
                  mask_v, rows_v, acc_v, sem):
    sid = lax.axis_index("s")
    wid = sid * NC + lax.axis_index("c")
    pair_base = wid * PAIRS_PER_W

    @pl.when(sid == 0)
    def _load_table():
        pltpu.sync_copy(table_hbm, table_sh)

    pltpu.sync_copy(idx_hbm.at[wid], idx_v)

    def mask_chunk(ci):
        pltpu.sync_copy(
            mask_hbm.at[pl.ds(wid * PAIRS_PER_W * TL + ci * IDX_CHUNK,
                              IDX_CHUNK)], mask_v)
        for j in range(NGATHER):
            for i in range(8):
                m = mask_v[pl.ds((j * 8 + i) * 16, 16)]
                iv = idx_v[ci * NGATHER + j, pl.ds(i * 16, 16)]
                idx_v[ci * NGATHER + j, pl.ds(i * 16, 16)] = (
                    jnp.where(m > 0.0, iv, V))

    def sel_body(ci, carry):
        mask_chunk(ci)
        return carry

    lax.fori_loop(0, NCHUNK, sel_body, 0)
    plsc.subcore_barrier()

    def chunk_body(ci, carry):
        pbase = pair_base + ci * CP
        copies = [
            pltpu.async_copy(table_sh.at[idx_v.at[ci * NGATHER + j]],
                             rows_v.at[pl.ds(j * 128, 128)], sem)
            for j in range(NGATHER)
        ]
        for c in copies:
            c.wait()

        def pair_body(p, c2):
            rb = p * TL
            himask = jnp.int32(-65536)
            for kk in range(WD // 32):
                def unpack2(w):
                    lo = lax.bitcast_convert_type(
                        lax.shift_left(w, 16), jnp.float32)
                    hi = lax.bitcast_convert_type(
                        lax.bitwise_and(w, himask), jnp.float32)
                    return lo, hi

                sa, sb = unpack2(rows_v[rb, pl.ds(kk * 16, 16)])
                for j in range(1, TL):
                    a, b = unpack2(rows_v[rb + j, pl.ds(kk * 16, 16)])
                    sa = sa + a
                    sb = sb + b
                acc_v[p, pl.ds(kk * 32, 16)] = sa
                acc_v[p, pl.ds(kk * 32 + 16, 16)] = sb
            return c2

        lax.fori_loop(0, CP, pair_body, 0)
        pltpu.sync_copy(acc_v, out_hbm.at[pl.ds(pbase, CP)])
        return carry

    lax.fori_loop(0, NCHUNK, chunk_body, 0)


@functools.cache
def _make_pool_sc():
    mesh = plsc.VectorSubcoreMesh(core_axis_name="c", subcore_axis_name="s")
    return pl.kernel(
        _pool_sc_body,
        mesh=mesh,
        out_type=jax.ShapeDtypeStruct((NPAIR, WD), jnp.float32),
        scratch_types=[
            pltpu.VMEM_SHARED((V + 1, WD // 2), jnp.int32),
            pltpu.VMEM((IDX_ROWS, 128), jnp.int32),
            pltpu.VMEM((IDX_CHUNK,), jnp.float32),
            pltpu.VMEM((IDX_CHUNK, WD // 2), jnp.int32),
            pltpu.VMEM((CP, WD), jnp.float32),
            pltpu.SemaphoreType.DMA,
        ],
        compiler_params=pltpu.CompilerParams(use_tc_tiling_on_sc=False),
    )


BB = 16
NHP = 56
NP = 80
CATPP = 24


def _dense_body(sums_ref, tmask_ref, gidx_ref, cmask_ref, graph_ref, cand_ref,
                Wn_ref, bn_ref, proxy_ref, W0_ref, b0_ref, W1_ref, b1_ref,
                Kw_ref, Qw_ref, Qb_ref, aW_ref, ab_ref, iKw_ref, iQw_ref,
                iQb_ref, out_ref):
    f32 = jnp.float32
    bf = jnp.bfloat16

    def mm(x, w):
        return lax.dot_general(x.astype(bf), w, (((1,), (0,)), ((), ())),
                               preferred_element_type=f32)

    cnt = jnp.sum(tmask_ref[...], axis=1, keepdims=True)
    pooled = sums_ref[...] / jnp.maximum(cnt, 1e-6)
    hist = mm(pooled, Wn_ref[...]) + bn_ref[...]
    proxy = proxy_ref[...]
    W0 = W0_ref[...]
    W1 = W1_ref[...]
    b0 = b0_ref[...]
    b1 = b1_ref[...]
    cand = cand_ref[...]

    h0s = [jnp.concatenate([hist[s * NHP:(s + 1) * NHP], proxy], axis=0)
           for s in range(BB)]
    H0 = jnp.concatenate(h0s, axis=0)
    T0 = jnp.concatenate(
        [lax.dot_general(graph_ref[s], h0s[s].astype(bf),
                         (((1,), (0,)), ((), ())), preferred_element_type=f32)
         for s in range(BB)], axis=0)
    H1 = jax.nn.relu(mm(T0, W0) + b0) + H0
    T1 = jnp.concatenate(
        [lax.dot_general(graph_ref[s], H1[s * NP:(s + 1) * NP].astype(bf),
                         (((1,), (0,)), ((), ())), preferred_element_type=f32)
         for s in range(BB)], axis=0)
    G = mm(T1, W1) + b1 + H1 + H0

    K = mm(G, Kw_ref[...]).astype(bf)
    Q = (mm(cand, Qw_ref[...]) + Qb_ref[...]).astype(bf)
    cat_iota = lax.broadcasted_iota(jnp.int32, (CATPP, NH), 0)

    a_list = []
    oh_list = []
    for s in range(BB):
        K_s = K[s * NP:s * NP + NH]
        Q_s = Q[s * NN:(s + 1) * NN]
        a_list.append(
            lax.dot_general(Q_s, K_s, (((1,), (1,)), ((), ())),
                            preferred_element_type=f32) / SCALE)
        oh_list.append(
            (cat_iota == gidx_ref[s][None, :]).astype(f32))
    A3 = jnp.stack(a_list)
    OH3 = jnp.stack(oh_list)
    SEGMAX = jnp.max(
        jnp.where(OH3[:, None, :, :] > 0, A3[:, :, None, :], -1e9), axis=3)
    MG3 = jnp.stack([SEGMAX[s] @ OH3[s] for s in range(BB)])
    EXPA = jnp.exp(A3 - MG3)
    DEN3 = jnp.stack(
        [lax.dot_general(EXPA[s], OH3[s], (((1,), (1,)), ((), ()))) @ OH3[s]
         for s in range(BB)])
    AL3 = EXPA / DEN3

    intras = []
    for s in range(BB):
        M = jnp.concatenate(
            [OH3[s] * AL3[s, n:n + 1, :] for n in range(NN)], axis=0)
        intras.append(
            lax.dot_general(M.astype(bf),
                            G[s * NP:s * NP + NH].astype(bf),
                            (((1,), (0,)), ((), ())),
                            preferred_element_type=f32))

    INTRA = jnp.concatenate(intras, axis=0)
    INTRA = jax.nn.relu(mm(INTRA, aW_ref[...]) + ab_ref[...]) + INTRA
    KF = mm(INTRA, iKw_ref[...])
    QF = mm(cand, iQw_ref[...]) + iQb_ref[...]

    KF3 = KF.reshape(BB * NN, CATPP, AD)
    satt = jnp.sum(KF3 * QF[:, None, :], axis=2) / SCALE
    satt = jnp.where(cmask_ref[...] == 0, -1e9, satt)
    satt = satt - jnp.max(satt, axis=1, keepdims=True)
    e = jnp.exp(satt)
    al = e / jnp.sum(e, axis=1, keepdims=True)
    out_ref[...] = jnp.sum(
        INTRA.reshape(BB * NN, CATPP, D) * al[:, :, None], axis=1)


def _full(shape):
    return pl.BlockSpec(shape, lambda i: (0,) * len(shape))


_dense = pl.pallas_call(
    _dense_body,
    grid=(B // BB,),
    in_specs=[
        pl.BlockSpec((BB * NHP, WD), lambda i: (i, 0)),
        pl.BlockSpec((BB * NHP, TL), lambda i: (i, 0)),
        pl.BlockSpec((BB, NH), lambda i: (i, 0)),
        pl.BlockSpec((BB * NN, CATPP), lambda i: (i, 0)),
        pl.BlockSpec((BB, NP, NP), lambda i: (i, 0, 0)),
        pl.BlockSpec((BB * NN, D), lambda i: (i, 0)),
        _full((WD, D)),
        _full((1, D)),
        _full((CATPP, D)),
        _full((D, D)),
        _full((1, D)),
        _full((D, D)),
        _full((1, D)),
        _full((D, AD)),
        _full((D, AD)),
        _full((1, AD)),
        _full((D, D)),
        _full((1, D)),
        _full((D, AD)),
        _full((D, AD)),
        _full((1, AD)),
    ],
    out_specs=pl.BlockSpec((BB * NN, D), lambda i: (i, 0)),
    out_shape=jax.ShapeDtypeStruct((B * NN, D), jnp.float32),
    compiler_params=pltpu.CompilerParams(
        dimension_semantics=("arbitrary",)),
)


def kernel(user_title_text, user_title_mask, user_title_entity,
           user_content_text, user_content_mask, user_content_entity,
           user_category, user_subCategory, user_history_mask,
           user_history_graph, user_history_category_mask,
           user_history_category_indices, user_embedding,
           candidate_news_representation, word_emb, W_news, b_news, proxy_emb,
           gcn_W0, gcn_b0, gcn_W1, gcn_b1, Kw, Qw, Qb, aff_W, aff_b, inter_Kw,
           inter_Qw, inter_Qb):
    idx2d = user_title_text.astype(jnp.int32).reshape(NW, IDX_ROWS, 128)
    mask1d = user_title_mask.reshape(-1)
    table_bf = jnp.concatenate(
        [word_emb.astype(jnp.bfloat16),
         jnp.zeros((1, WD), jnp.bfloat16)], axis=0)
    table_z = lax.bitcast_convert_type(
        table_bf.reshape(V + 1, WD // 2, 2), jnp.int32)
    sums = _make_pool_sc()(idx2d, mask1d, table_z)

    sums_p = jnp.pad(sums.reshape(B, NH, WD),
                     ((0, 0), (0, NHP - NH), (0, 0))).reshape(B * NHP, WD)
    tmask_p = jnp.pad(user_title_mask,
                      ((0, 0), (0, NHP - NH), (0, 0))).reshape(B * NHP, TL)
    Ag = user_history_graph
    zc = jnp.zeros((B, NH, NHP - NH), jnp.float32)
    zc2 = jnp.zeros((B, CAT, NHP - NH), jnp.float32)
    top = jnp.concatenate(
        [Ag[:, :NH, :NH], zc, Ag[:, :NH, NH:], zc], axis=2)
    bot = jnp.concatenate(
        [Ag[:, NH:, :NH], zc2, Ag[:, NH:, NH:], zc2], axis=2)
    graph_p = jnp.concatenate(
        [top, jnp.zeros((B, NHP - NH, NP), jnp.float32), bot,
         jnp.zeros((B, NP - NHP - CAT, NP), jnp.float32)], axis=1)
    proxy_p = jnp.pad(proxy_emb, ((0, CATPP - CAT + 1), (0, 0)))[:CATPP]
    cmask_p = jnp.repeat(
        jnp.pad(user_history_category_mask.at[:, -1].set(1.0),
                ((0, 0), (0, CATPP - CATP))), NN, axis=0)
    cand2 = candidate_news_representation.reshape(B * NN, D)

    bf = jnp.bfloat16
    out = _dense(
        sums_p,
        tmask_p,
        user_history_category_indices.astype(jnp.int32),
        cmask_p,
        graph_p.astype(bf),
        cand2,
        W_news[jnp.array(_UNPACK_PERM)].astype(bf),
        b_news.reshape(1, D),
        proxy_p,
        gcn_W0.astype(bf),
        gcn_b0.reshape(1, D),
        gcn_W1.astype(bf),
        gcn_b1.reshape(1, D),
        Kw.astype(bf),
        Qw.astype(bf),
        Qb.reshape(1, AD),
        aff_W.astype(bf),
        aff_b.reshape(1, D),
        inter_Kw.astype(bf),
        inter_Qw.astype(bf),
        inter_Qb.reshape(1, AD),
    )
    return out.reshape(B, NN, D)
